# Initial kernel scaffold; baseline (speedup 1.0000x reference)
import jax
import jax.numpy as jnp
from jax.experimental import pallas as pl


def kernel(rssi_seq, ap_emb, rank_template):
    raise NotImplementedError("write your pallas kernel here")


# speedup vs baseline: 4.8286x; 4.8286x over previous
import functools

import jax
import jax.numpy as jnp
from jax import lax
from jax.experimental import pallas as pl
from jax.experimental.pallas import tpu as pltpu
from jax.experimental.pallas import tpu_sc as plsc

_K = 32
_EMB_PAD = 16


def _topk_body(x_ref, vals_ref, idx_ref, delta_ref, isnew_ref):
    x = x_ref[0]
    t_dim, n_dim = x.shape
    prev = jnp.concatenate([x[:1, :], x[:-1, :]], axis=0)
    iota = lax.broadcasted_iota(jnp.int32, (t_dim, n_dim), 1)
    col = lax.broadcasted_iota(jnp.int32, (t_dim, _K), 1)

    def step(j, carry):
        xx, vals, idxs, pvs = carry
        m = jnp.max(xx, axis=1, keepdims=True)
        sel = jnp.min(jnp.where(xx == m, iota, n_dim), axis=1, keepdims=True)
        onehot = iota == sel
        pv = jnp.sum(jnp.where(onehot, prev, 0.0), axis=1, keepdims=True)
        vals = jnp.where(col == j, m, vals)
        idxs = jnp.where(col == j, sel, idxs)
        pvs = jnp.where(col == j, pv, pvs)
        xx = jnp.where(onehot, -1.0, xx)
        return xx, vals, idxs, pvs

    init = (
        x,
        jnp.zeros((t_dim, _K), jnp.float32),
        jnp.zeros((t_dim, _K), jnp.int32),
        jnp.zeros((t_dim, _K), jnp.float32),
    )
    _, vals, idxs, pvs = lax.fori_loop(0, _K, step, init)
    vals_ref[0] = vals
    idx_ref[0] = idxs
    delta_ref[0] = vals - pvs
    isnew_ref[0] = ((pvs <= 1e-6) & (vals > 1e-6)).astype(jnp.float32)


def _topk_call(rssi):
    b_dim, t_dim, n_dim = rssi.shape
    ospec = pl.BlockSpec((1, t_dim, _K), lambda i: (i, 0, 0))
    return pl.pallas_call(
        _topk_body,
        grid=(b_dim,),
        in_specs=[pl.BlockSpec((1, t_dim, n_dim), lambda i: (i, 0, 0))],
        out_specs=[ospec, ospec, ospec, ospec],
        out_shape=[
            jax.ShapeDtypeStruct((b_dim, t_dim, _K), jnp.float32),
            jax.ShapeDtypeStruct((b_dim, t_dim, _K), jnp.int32),
            jax.ShapeDtypeStruct((b_dim, t_dim, _K), jnp.float32),
            jax.ShapeDtypeStruct((b_dim, t_dim, _K), jnp.float32),
        ],
    )(rssi)


def _emb_gather_call(emb_pad, idx_flat):
    info = plsc.get_sparse_core_info()
    nc, ns = info.num_cores, info.num_subcores
    nw = nc * ns
    n_tok = idx_flat.shape[0]
    bpw = n_tok // nw

    mesh = plsc.VectorSubcoreMesh(core_axis_name="c", subcore_axis_name="s")

    @functools.partial(
        pl.kernel,
        mesh=mesh,
        compiler_params=pltpu.CompilerParams(use_tc_tiling_on_sc=False),
        out_type=jax.ShapeDtypeStruct((n_tok, _EMB_PAD), jnp.float32),
        scratch_types=[
            pltpu.VMEM((bpw,), jnp.int32),
            pltpu.VMEM((bpw, _EMB_PAD), jnp.float32),
            pltpu.SemaphoreType.DMA,
        ],
    )
    def k(emb_hbm, idx_hbm, out_hbm, idx_v, rows_v, sem):
        wid = lax.axis_index("s") * nc + lax.axis_index("c")
        base = wid * bpw
        pltpu.sync_copy(idx_hbm.at[pl.ds(base, bpw)], idx_v)
        pltpu.async_copy(emb_hbm.at[idx_v], rows_v, sem).wait()
        pltpu.sync_copy(rows_v, out_hbm.at[pl.ds(base, bpw)])

    return k(emb_pad, idx_flat)


def kernel(rssi_seq, ap_emb, rank_template):
    b_dim, t_dim, _ = rssi_seq.shape
    emb_dim = ap_emb.shape[1]
    vals, idxs, delta, isnew = _topk_call(rssi_seq)
    emb_pad = jnp.pad(ap_emb, ((0, 0), (0, _EMB_PAD - emb_dim)))
    feat = _emb_gather_call(emb_pad, idxs.reshape(-1))
    ap_feat = feat[:, :emb_dim].reshape(b_dim, t_dim, _K, emb_dim)
    rank = jnp.broadcast_to(rank_template[:, :, :_K, :], (b_dim, t_dim, _K, 1))
    return jnp.concatenate(
        [ap_feat, vals[..., None], delta[..., None], rank, isnew[..., None]],
        axis=-1,
    )


# speedup vs baseline: 7.7695x; 1.6091x over previous
import functools

import jax
import jax.numpy as jnp
from jax import lax
from jax.experimental import pallas as pl
from jax.experimental.pallas import tpu as pltpu
from jax.experimental.pallas import tpu_sc as plsc

_K = 32
_EMB_PAD = 16
_N = 8192
_NV = _N // 16
_BIG = 1 << 30
_TOK_W = 12


def _sc_call(rssi_flat, emb_pad, rank_flat, n_rows):
    info = plsc.get_sparse_core_info()
    nc, ns = info.num_cores, info.num_subcores
    nw = nc * ns
    rpw = n_rows // nw

    mesh = plsc.VectorSubcoreMesh(core_axis_name="c", subcore_axis_name="s")

    @functools.partial(
        pl.kernel,
        mesh=mesh,
        compiler_params=pltpu.CompilerParams(use_tc_tiling_on_sc=False,
                                             needs_layout_passes=False),
        out_type=jax.ShapeDtypeStruct((n_rows * _K * _TOK_W,), jnp.float32),
        scratch_types=[
            pltpu.VMEM((_N,), jnp.float32),
            pltpu.VMEM((_N,), jnp.float32),
            pltpu.VMEM((_N + 16,), jnp.float32),
            pltpu.VMEM((_N + 16,), jnp.int32),
            pltpu.VMEM((_K,), jnp.float32),
            pltpu.VMEM((_K,), jnp.int32),
            pltpu.VMEM((_K, _EMB_PAD), jnp.float32),
            pltpu.VMEM((_K * _TOK_W,), jnp.float32),
            pltpu.VMEM((_K,), jnp.float32),
            pltpu.SemaphoreType.DMA,
        ],
    )
    def k(rssi_hbm, emb_hbm, rank_hbm, out_hbm,
          row_a, row_b, cand_v, cand_i, topv, topi, embbuf, tokbuf, rankbuf,
          sem):
        wid = lax.axis_index("s") * nc + lax.axis_index("c")
        base_row = wid * rpw
        iota16 = lax.iota(jnp.int32, 16)
        pltpu.sync_copy(rank_hbm, rankbuf)

        def process_row(row, cur, prev, is_t0):
            def amax(c, m):
                return jnp.maximum(m, cur[pl.ds(c * 16, 16)])

            neg1 = jnp.full((16,), -1.0, jnp.float32)
            m1 = lax.fori_loop(0, _NV // 2, amax, neg1)
            m2 = lax.fori_loop(_NV // 2, _NV, amax, neg1)
            thr = jnp.min(jnp.minimum(m1, m2))

            def collect(c, cnt):
                v = cur[pl.ds(c * 16, 16)]
                msk = v >= thr
                plsc.store_compressed(cand_v.at[pl.ds(cnt, 16)], v, mask=msk)
                plsc.store_compressed(
                    cand_i.at[pl.ds(cnt, 16)], iota16 + c * 16, mask=msk)
                return cnt + plsc.all_reduce_population_count(msk)[0]

            cnt = lax.fori_loop(0, _NV, collect, jnp.int32(0))
            cand_v[pl.ds(cnt, 16)] = neg1
            nv = (cnt + 15) // 16

            def extract(j, _):
                def p1(v, m):
                    return jnp.maximum(m, cand_v[pl.ds(v * 16, 16)])

                m = jnp.max(lax.fori_loop(0, nv, p1, neg1))

                def p2(v, b):
                    cv = cand_v[pl.ds(v * 16, 16)]
                    ci = cand_i[pl.ds(v * 16, 16)]
                    return jnp.minimum(b, jnp.where(cv == m, ci, _BIG))

                b = jnp.min(lax.fori_loop(
                    0, nv, p2, jnp.full((16,), _BIG, jnp.int32)))
                lane0 = iota16 == 0
                jsplat = jnp.full((16,), j, jnp.int32)
                plsc.store_scatter(topv, [jsplat], jnp.full((16,), m),
                                   mask=lane0)
                plsc.store_scatter(topi, [jsplat], jnp.full((16,), b),
                                   mask=lane0)

                def p3(v, _):
                    sl = pl.ds(v * 16, 16)
                    cv = cand_v[sl]
                    ci = cand_i[sl]
                    cand_v[sl] = jnp.where((cv == m) & (ci == b), -1.0, cv)
                    return 0

                lax.fori_loop(0, nv, p3, 0)
                return 0

            lax.fori_loop(0, _K, extract, 0)

            pltpu.async_copy(emb_hbm.at[topi], embbuf, sem).wait()
            for h in range(2):
                sl = pl.ds(h * 16, 16)
                tv = topv[sl]
                ti = topi[sl]
                pv = jnp.where(is_t0, tv, plsc.load_gather(prev, [ti]))
                delta = tv - pv
                isnew = jnp.where((pv <= 1e-6) & (tv > 1e-6), 1.0, 0.0)
                rk = rankbuf[sl]
                tok = iota16 + h * 16
                tgt = tok * _TOK_W
                plsc.store_scatter(tokbuf, [tgt + 8], tv)
                plsc.store_scatter(tokbuf, [tgt + 9], delta)
                plsc.store_scatter(tokbuf, [tgt + 10], rk)
                plsc.store_scatter(tokbuf, [tgt + 11], isnew)
                for d in range(8):
                    ev = plsc.load_gather(
                        embbuf, [tok, jnp.full((16,), d, jnp.int32)])
                    plsc.store_scatter(tokbuf, [tgt + d], ev)
            pltpu.sync_copy(tokbuf,
                            out_hbm.at[pl.ds(row * (_K * _TOK_W),
                                             _K * _TOK_W)])

        def pair(i, _):
            r0 = base_row + 2 * i
            pltpu.sync_copy(rssi_hbm.at[pl.ds(r0 * _N, _N)], row_a)
            process_row(r0, row_a, row_b, (2 * i) % 32 == 0)
            r1 = r0 + 1
            pltpu.sync_copy(rssi_hbm.at[pl.ds(r1 * _N, _N)], row_b)
            process_row(r1, row_b, row_a, False)
            return 0

        lax.fori_loop(0, rpw // 2, pair, 0)

    return k(rssi_flat, emb_pad, rank_flat)


def kernel(rssi_seq, ap_emb, rank_template):
    b_dim, t_dim, _ = rssi_seq.shape
    emb_dim = ap_emb.shape[1]
    n_rows = b_dim * t_dim
    emb_pad = jnp.pad(ap_emb, ((0, 0), (0, _EMB_PAD - emb_dim)))
    out = _sc_call(rssi_seq.reshape(-1), emb_pad,
                   rank_template.reshape(-1), n_rows)
    return out.reshape(b_dim, t_dim, _K, _TOK_W)


# speedup vs baseline: 19.4158x; 2.4990x over previous
import functools

import jax
import jax.numpy as jnp
from jax import lax
from jax.experimental import pallas as pl
from jax.experimental.pallas import tpu as pltpu
from jax.experimental.pallas import tpu_sc as plsc

_K = 32
_EMB_PAD = 16
_N = 8192
_NV = _N // 16
_BIG = 1 << 30
_TOK_W = 12


def _sc_call(rssi_flat, emb_pad, rank_flat, n_rows):
    info = plsc.get_sparse_core_info()
    nc, ns = info.num_cores, info.num_subcores
    nw = nc * ns
    rpw = n_rows // nw

    mesh = plsc.VectorSubcoreMesh(core_axis_name="c", subcore_axis_name="s")

    @functools.partial(
        pl.kernel,
        mesh=mesh,
        compiler_params=pltpu.CompilerParams(use_tc_tiling_on_sc=False,
                                             needs_layout_passes=False),
        out_type=jax.ShapeDtypeStruct((n_rows * _K * _TOK_W,), jnp.float32),
        scratch_types=[
            pltpu.VMEM((_N,), jnp.float32),
            pltpu.VMEM((_N,), jnp.float32),
            pltpu.VMEM((_N + 16,), jnp.float32),
            pltpu.VMEM((_N + 16,), jnp.int32),
            pltpu.VMEM((_K,), jnp.float32),
            pltpu.VMEM((_K,), jnp.int32),
            pltpu.VMEM((_K, _EMB_PAD), jnp.float32),
            pltpu.VMEM((_K * _TOK_W,), jnp.float32),
            pltpu.VMEM((_K,), jnp.float32),
            pltpu.SemaphoreType.DMA,
        ],
    )
    def k(rssi_hbm, emb_hbm, rank_hbm, out_hbm,
          row_a, row_b, cand_v, cand_i, topv, topi, embbuf, tokbuf, rankbuf,
          sem):
        wid = lax.axis_index("s") * nc + lax.axis_index("c")
        base_row = wid * rpw
        iota16 = lax.iota(jnp.int32, 16)
        pltpu.sync_copy(rank_hbm, rankbuf)

        lane0 = iota16 == 0

        def emit_top(j, m, b):
            jsplat = jnp.full((16,), j, jnp.int32)
            plsc.store_scatter(topv, [jsplat], jnp.full((16,), m), mask=lane0)
            plsc.store_scatter(topi, [jsplat], jnp.full((16,), b), mask=lane0)

        def process_row(row, cur, prev, is_t0):
            neg1 = jnp.full((16,), -1.0, jnp.float32)

            @plsc.parallel_loop(0, _NV // 2, unroll=8, carry=(neg1, neg1))
            def amax(c, ms):
                m1, m2 = ms
                return (jnp.maximum(m1, cur[pl.ds(c * 16, 16)]),
                        jnp.maximum(m2, cur[pl.ds((c + _NV // 2) * 16, 16)]))

            m1, m2 = amax
            thr = jnp.min(jnp.minimum(m1, m2))

            @plsc.parallel_loop(0, _NV, unroll=8, carry=jnp.int32(0))
            def collect(c, cnt):
                v = cur[pl.ds(c * 16, 16)]
                msk = v >= thr
                plsc.store_compressed(cand_v.at[pl.ds(cnt, 16)], v, mask=msk)
                plsc.store_compressed(
                    cand_i.at[pl.ds(cnt, 16)], iota16 + c * 16, mask=msk)
                return cnt + plsc.all_reduce_population_count(msk)[0]

            cnt = collect
            cand_v[pl.ds(cnt, 16)] = neg1

            @pl.when(cnt <= 256)
            def _fast():
                summ = jnp.full((16,), -1.0, jnp.float32)
                for v in range(16):
                    sl = pl.ds(v * 16, 16)
                    cv = jnp.where(iota16 + v * 16 < cnt, cand_v[sl], -1.0)
                    cand_v[sl] = cv
                    summ = jnp.where(iota16 == v, jnp.max(cv), summ)

                def ext(j, summ):
                    m = jnp.max(summ)
                    bv = plsc.all_reduce_ffs(summ == m)[0]
                    sl = pl.ds(bv * 16, 16)
                    cv = cand_v[sl]
                    ci = cand_i[sl]
                    eqm = cv == m
                    b = jnp.min(jnp.where(eqm, ci, _BIG))
                    emit_top(j, m, b)
                    cv = jnp.where(eqm & (ci == b), -1.0, cv)
                    cand_v[sl] = cv
                    return jnp.where(iota16 == bv, jnp.max(cv), summ)

                lax.fori_loop(0, _K, ext, summ)

            @pl.when(cnt > 256)
            def _slow():
                nv = (cnt + 15) // 16

                def extract(j, _):
                    def p1(v, m):
                        return jnp.maximum(m, cand_v[pl.ds(v * 16, 16)])

                    m = jnp.max(lax.fori_loop(0, nv, p1, neg1))

                    def p2(v, b):
                        cv = cand_v[pl.ds(v * 16, 16)]
                        ci = cand_i[pl.ds(v * 16, 16)]
                        return jnp.minimum(b, jnp.where(cv == m, ci, _BIG))

                    b = jnp.min(lax.fori_loop(
                        0, nv, p2, jnp.full((16,), _BIG, jnp.int32)))
                    emit_top(j, m, b)

                    def p3(v, _):
                        sl = pl.ds(v * 16, 16)
                        cv = cand_v[sl]
                        ci = cand_i[sl]
                        cand_v[sl] = jnp.where((cv == m) & (ci == b), -1.0, cv)
                        return 0

                    lax.fori_loop(0, nv, p3, 0)
                    return 0

                lax.fori_loop(0, _K, extract, 0)

            pltpu.async_copy(emb_hbm.at[topi], embbuf, sem).wait()
            for h in range(2):
                sl = pl.ds(h * 16, 16)
                tv = topv[sl]
                ti = topi[sl]
                pv = jnp.where(is_t0, tv, plsc.load_gather(prev, [ti]))
                delta = tv - pv
                isnew = jnp.where((pv <= 1e-6) & (tv > 1e-6), 1.0, 0.0)
                rk = rankbuf[sl]
                tok = iota16 + h * 16
                tgt = tok * _TOK_W
                plsc.store_scatter(tokbuf, [tgt + 8], tv)
                plsc.store_scatter(tokbuf, [tgt + 9], delta)
                plsc.store_scatter(tokbuf, [tgt + 10], rk)
                plsc.store_scatter(tokbuf, [tgt + 11], isnew)
                for d in range(8):
                    ev = plsc.load_gather(
                        embbuf, [tok, jnp.full((16,), d, jnp.int32)])
                    plsc.store_scatter(tokbuf, [tgt + d], ev)
            pltpu.sync_copy(tokbuf,
                            out_hbm.at[pl.ds(row * (_K * _TOK_W),
                                             _K * _TOK_W)])

        def pair(i, _):
            r0 = base_row + 2 * i
            pltpu.sync_copy(rssi_hbm.at[pl.ds(r0 * _N, _N)], row_a)
            process_row(r0, row_a, row_b, (2 * i) % 32 == 0)
            r1 = r0 + 1
            pltpu.sync_copy(rssi_hbm.at[pl.ds(r1 * _N, _N)], row_b)
            process_row(r1, row_b, row_a, False)
            return 0

        lax.fori_loop(0, rpw // 2, pair, 0)

    return k(rssi_flat, emb_pad, rank_flat)


def kernel(rssi_seq, ap_emb, rank_template):
    b_dim, t_dim, _ = rssi_seq.shape
    emb_dim = ap_emb.shape[1]
    n_rows = b_dim * t_dim
    emb_pad = jnp.pad(ap_emb, ((0, 0), (0, _EMB_PAD - emb_dim)))
    out = _sc_call(rssi_seq.reshape(-1), emb_pad,
                   rank_template.reshape(-1), n_rows)
    return out.reshape(b_dim, t_dim, _K, _TOK_W)


# speedup vs baseline: 23.4906x; 1.2099x over previous
import functools

import jax
import jax.numpy as jnp
from jax import lax
from jax.experimental import pallas as pl
from jax.experimental.pallas import tpu as pltpu
from jax.experimental.pallas import tpu_sc as plsc

_K = 32
_EMB_PAD = 16
_N = 8192
_NV = _N // 16
_BIG = 1 << 30
_TOK_W = 12


def _sc_call(rssi_flat, emb_pad, rank_flat, n_rows):
    info = plsc.get_sparse_core_info()
    nc, ns = info.num_cores, info.num_subcores
    nw = nc * ns
    rpw = n_rows // nw

    mesh = plsc.VectorSubcoreMesh(core_axis_name="c", subcore_axis_name="s")

    @functools.partial(
        pl.kernel,
        mesh=mesh,
        compiler_params=pltpu.CompilerParams(use_tc_tiling_on_sc=False,
                                             needs_layout_passes=False),
        out_type=jax.ShapeDtypeStruct((n_rows * _K * _TOK_W,), jnp.float32),
        scratch_types=[
            pltpu.VMEM((_N,), jnp.float32),
            pltpu.VMEM((_N,), jnp.float32),
            pltpu.VMEM((_N + 16,), jnp.float32),
            pltpu.VMEM((_N + 16,), jnp.int32),
            pltpu.VMEM((_K,), jnp.float32),
            pltpu.VMEM((_K,), jnp.int32),
            pltpu.VMEM((_K, _EMB_PAD), jnp.float32),
            pltpu.VMEM((_K * _TOK_W,), jnp.float32),
            pltpu.VMEM((_K,), jnp.float32),
            pltpu.SemaphoreType.DMA,
        ],
    )
    def k(rssi_hbm, emb_hbm, rank_hbm, out_hbm,
          row_a, row_b, cand_v, cand_i, topv, topi, embbuf, tokbuf, rankbuf,
          sem):
        wid = lax.axis_index("s") * nc + lax.axis_index("c")
        base_row = wid * rpw
        iota16 = lax.iota(jnp.int32, 16)
        pltpu.sync_copy(rank_hbm, rankbuf)

        lane0 = iota16 == 0

        def emit_top(j, m, b):
            jsplat = jnp.full((16,), j, jnp.int32)
            plsc.store_scatter(topv, [jsplat],
                               jnp.broadcast_to(m, (16,)), mask=lane0)
            plsc.store_scatter(topi, [jsplat],
                               jnp.broadcast_to(b, (16,)), mask=lane0)

        def process_row(row, cur, prev, is_t0, t_spec):
            neg1 = jnp.full((16,), -1.0, jnp.float32)

            def collect_idx(thr):
                @plsc.parallel_loop(0, _NV, unroll=8, carry=jnp.int32(0))
                def collect(c, cnt):
                    msk = cur[pl.ds(c * 16, 16)] >= thr
                    plsc.store_compressed(
                        cand_i.at[pl.ds(cnt, 16)], iota16 + c * 16, mask=msk)
                    return cnt + plsc.all_reduce_population_count(msk)[0]

                return collect

            def exact_thr():
                @plsc.parallel_loop(0, _NV // 2, unroll=8, carry=(neg1, neg1))
                def amax(c, ms):
                    m1, m2 = ms
                    return (jnp.maximum(m1, cur[pl.ds(c * 16, 16)]),
                            jnp.maximum(m2,
                                        cur[pl.ds((c + _NV // 2) * 16, 16)]))

                m1, m2 = amax
                return jnp.min(jnp.minimum(m1, m2))

            cnt0 = collect_idx(t_spec)
            cnt = lax.cond((cnt0 >= _K) & (cnt0 <= 256),
                           lambda: cnt0,
                           lambda: collect_idx(exact_thr()))

            @pl.when(cnt <= 256)
            def _fast():
                summ = jnp.full((16,), -1.0, jnp.float32)
                for v in range(16):
                    sl = pl.ds(v * 16, 16)
                    valid = iota16 + v * 16 < cnt
                    cv = plsc.load_gather(cur, [cand_i[sl]], mask=valid)
                    cv = jnp.where(valid, cv, -1.0)
                    cand_v[sl] = cv
                    summ = jnp.where(iota16 == v, jnp.max(cv), summ)

                def ext(j, summ):
                    m = jnp.max(summ)
                    bv = plsc.all_reduce_ffs(summ == m)[0]
                    sl = pl.ds(bv * 16, 16)
                    cv = cand_v[sl]
                    l0 = plsc.all_reduce_ffs(cv == m)
                    b = cand_i[sl].at[l0].get(mode="promise_in_bounds")
                    emit_top(j, jnp.full((16,), m), b)
                    cv = jnp.where(iota16 == l0, -1.0, cv)
                    cand_v[sl] = cv
                    return jnp.where(iota16 == bv, jnp.max(cv), summ)

                lax.fori_loop(0, _K, ext, summ)

            @pl.when(cnt > 256)
            def _slow():
                nv = (cnt + 15) // 16

                def mat(v, _):
                    sl = pl.ds(v * 16, 16)
                    valid = iota16 + v * 16 < cnt
                    cv = plsc.load_gather(cur, [cand_i[sl]], mask=valid)
                    cand_v[sl] = jnp.where(valid, cv, -1.0)
                    return 0

                lax.fori_loop(0, nv, mat, 0)

                def extract(j, _):
                    def p1(v, m):
                        return jnp.maximum(m, cand_v[pl.ds(v * 16, 16)])

                    m = jnp.max(lax.fori_loop(0, nv, p1, neg1))

                    def p2(v, b):
                        cv = cand_v[pl.ds(v * 16, 16)]
                        ci = cand_i[pl.ds(v * 16, 16)]
                        return jnp.minimum(b, jnp.where(cv == m, ci, _BIG))

                    b = jnp.min(lax.fori_loop(
                        0, nv, p2, jnp.full((16,), _BIG, jnp.int32)))
                    emit_top(j, m, b)

                    def p3(v, _):
                        sl = pl.ds(v * 16, 16)
                        cv = cand_v[sl]
                        ci = cand_i[sl]
                        cand_v[sl] = jnp.where((cv == m) & (ci == b), -1.0, cv)
                        return 0

                    lax.fori_loop(0, nv, p3, 0)
                    return 0

                lax.fori_loop(0, _K, extract, 0)

            dma = pltpu.async_copy(emb_hbm.at[topi], embbuf, sem)
            for h in range(2):
                sl = pl.ds(h * 16, 16)
                tv = topv[sl]
                ti = topi[sl]
                pv = jnp.where(is_t0, tv, plsc.load_gather(prev, [ti]))
                delta = tv - pv
                isnew = jnp.where((pv <= 1e-6) & (tv > 1e-6), 1.0, 0.0)
                rk = rankbuf[sl]
                tgt = (iota16 + h * 16) * _TOK_W
                plsc.store_scatter(tokbuf, [tgt + 8], tv)
                plsc.store_scatter(tokbuf, [tgt + 9], delta)
                plsc.store_scatter(tokbuf, [tgt + 10], rk)
                plsc.store_scatter(tokbuf, [tgt + 11], isnew)
            dma.wait()
            for h in range(2):
                tok = iota16 + h * 16
                tgt = tok * _TOK_W
                for d in range(8):
                    ev = plsc.load_gather(
                        embbuf, [tok, jnp.full((16,), d, jnp.int32)])
                    plsc.store_scatter(tokbuf, [tgt + d], ev)
            pltpu.sync_copy(tokbuf,
                            out_hbm.at[pl.ds(row * (_K * _TOK_W),
                                             _K * _TOK_W)])
            return 2.0 * topv[pl.ds(_K - 16, 16)][15] - 1.0

        def pair(i, t_spec):
            r0 = base_row + 2 * i
            pltpu.sync_copy(rssi_hbm.at[pl.ds(r0 * _N, _N)], row_a)
            t_spec = process_row(r0, row_a, row_b, (2 * i) % 32 == 0, t_spec)
            r1 = r0 + 1
            pltpu.sync_copy(rssi_hbm.at[pl.ds(r1 * _N, _N)], row_b)
            return process_row(r1, row_b, row_a, False, t_spec)

        lax.fori_loop(0, rpw // 2, pair, jnp.float32(2.0))

    return k(rssi_flat, emb_pad, rank_flat)


def kernel(rssi_seq, ap_emb, rank_template):
    b_dim, t_dim, _ = rssi_seq.shape
    emb_dim = ap_emb.shape[1]
    n_rows = b_dim * t_dim
    emb_pad = jnp.pad(ap_emb, ((0, 0), (0, _EMB_PAD - emb_dim)))
    out = _sc_call(rssi_seq.reshape(-1), emb_pad,
                   rank_template.reshape(-1), n_rows)
    return out.reshape(b_dim, t_dim, _K, _TOK_W)
